# baseline (device time: 108735 ns/iter reference)
import jax
import jax.numpy as jnp
from jax import lax
from jax.experimental import pallas as pl
from jax.experimental.pallas import tpu as pltpu

N_DEV = 4


def kernel(x, Win0, Wout0, Win1, Wout1, Win2, Wout2):
    m_per, d = x.shape
    M = N_DEV * m_per

    def body(x_ref, win0, wout0, win1, wout1, win2, wout2,
             out_ref, xfull, p_ref, comm, send_sems, recv_sems):
        my = lax.axis_index("i")
        left = (my - 1) % N_DEV
        right = (my + 1) % N_DEV

        barrier_sem = pltpu.get_barrier_semaphore()
        for nbr in (left, right):
            pl.semaphore_signal(barrier_sem, inc=1, device_id=(nbr,),
                                device_id_type=pl.DeviceIdType.MESH)
        pl.semaphore_wait(barrier_sem, 2)

        def hop(H):
            s, r = H % 2, (H + 1) % 2
            rdma = pltpu.make_async_remote_copy(
                src_ref=comm.at[s],
                dst_ref=comm.at[r],
                send_sem=send_sems.at[s],
                recv_sem=recv_sems.at[r],
                device_id=(right,),
                device_id_type=pl.DeviceIdType.MESH,
            )
            rdma.start()
            rdma.wait()
            return r

        H = 0

        xfull[pl.ds(my * m_per, m_per), :] = x_ref[:, :]
        comm[0, :, :] = x_ref[:, :]
        for h in range(N_DEV - 1):
            r = hop(H); H += 1
            origin = (my - h - 1) % N_DEV
            xfull[pl.ds(origin * m_per, m_per), :] = comm[r, :, :]

        for win, wout, dest in ((win0, wout0, xfull),
                                (win1, wout1, xfull),
                                (win2, wout2, out_ref)):
            hmat = jnp.maximum(
                jnp.dot(xfull[:, :], win[:, :],
                        preferred_element_type=jnp.float32), 0.0)
            p_ref[:, :] = jnp.dot(hmat, wout[:, :],
                                  preferred_element_type=jnp.float32)

            comm[H % 2, :, :] = p_ref[pl.ds(my * m_per, m_per), :]
            for s in range(N_DEV - 1):
                r = hop(H); H += 1
                c = (my - s - 1) % N_DEV
                comm[r, :, :] = comm[r, :, :] + p_ref[pl.ds(c * m_per, m_per), :]
            own = (my + 1) % N_DEV
            dest[pl.ds(own * m_per, m_per), :] = comm[r, :, :]

            for s in range(N_DEV - 1):
                r = hop(H); H += 1
                c = (my - s) % N_DEV
                dest[pl.ds(c * m_per, m_per), :] = comm[r, :, :]

    return pl.pallas_call(
        body,
        out_shape=jax.ShapeDtypeStruct((M, d), jnp.float32),
        in_specs=[pl.BlockSpec(memory_space=pltpu.VMEM)] * 7,
        out_specs=pl.BlockSpec(memory_space=pltpu.VMEM),
        scratch_shapes=[
            pltpu.VMEM((M, d), jnp.float32),
            pltpu.VMEM((M, d), jnp.float32),
            pltpu.VMEM((2, m_per, d), jnp.float32),
            pltpu.SemaphoreType.DMA((2,)),
            pltpu.SemaphoreType.DMA((2,)),
        ],
        compiler_params=pltpu.CompilerParams(collective_id=0),
    )(x, Win0, Wout0, Win1, Wout1, Win2, Wout2)


# device time: 62291 ns/iter; 1.7456x vs baseline; 1.7456x over previous
import jax
import jax.numpy as jnp
from jax import lax
from jax.experimental import pallas as pl
from jax.experimental.pallas import tpu as pltpu

N_DEV = 4
DST_SLOT = (1, 0, 2)


def kernel(x, Win0, Wout0, Win1, Wout1, Win2, Wout2):
    m_per, d = x.shape
    M = N_DEV * m_per

    def body(x_ref, win0, wout0, win1, wout1, win2, wout2,
             out_ref, xown, xin, pbuf, rsbuf,
             ag_rsems, rs_rsems, ag_ssems, rs_ssems):
        my = lax.axis_index("i")
        peers = ((my + 1) % N_DEV, (my + 3) % N_DEV, (my + 2) % N_DEV)

        barrier_sem = pltpu.get_barrier_semaphore()
        for j in peers:
            pl.semaphore_signal(barrier_sem, inc=1, device_id=(j,),
                                device_id_type=pl.DeviceIdType.MESH)
        pl.semaphore_wait(barrier_sem, N_DEV - 1)

        def ag_send(s):
            return pltpu.make_async_remote_copy(
                src_ref=xown, dst_ref=xin.at[DST_SLOT[s]],
                send_sem=ag_ssems.at[s], recv_sem=ag_rsems.at[DST_SLOT[s]],
                device_id=(peers[s],), device_id_type=pl.DeviceIdType.MESH)

        def ag_wait_recv(s):
            pltpu.make_async_remote_copy(
                src_ref=xown, dst_ref=xin.at[s],
                send_sem=ag_ssems.at[s], recv_sem=ag_rsems.at[s],
                device_id=(peers[s],), device_id_type=pl.DeviceIdType.MESH,
            ).wait_recv()

        def rs_send(s):
            return pltpu.make_async_remote_copy(
                src_ref=pbuf.at[s], dst_ref=rsbuf.at[DST_SLOT[s]],
                send_sem=rs_ssems.at[s], recv_sem=rs_rsems.at[DST_SLOT[s]],
                device_id=(peers[s],), device_id_type=pl.DeviceIdType.MESH)

        def rs_wait_recv(s):
            pltpu.make_async_remote_copy(
                src_ref=pbuf.at[s], dst_ref=rsbuf.at[s],
                send_sem=rs_ssems.at[s], recv_sem=rs_rsems.at[s],
                device_id=(peers[s],), device_id_type=pl.DeviceIdType.MESH,
            ).wait_recv()

        xown[:, :] = x_ref[:, :]
        ag_flights = [ag_send(s) for s in range(3)]
        for f in ag_flights:
            f.start()

        rs_flights_prev = None
        for li, (win, wout) in enumerate(((win0, wout0), (win1, wout1),
                                          (win2, wout2))):
            p_own = jnp.dot(
                jnp.maximum(jnp.dot(xown[:, :], win[:, :],
                                    preferred_element_type=jnp.float32), 0.0),
                wout[:, :], preferred_element_type=jnp.float32)
            rs_flights = []
            for s in range(3):
                ag_wait_recv(s)
                if li > 0:
                    rs_flights_prev[s].wait_send()
                pbuf[s] = jnp.dot(
                    jnp.maximum(jnp.dot(xin[s], win[:, :],
                                        preferred_element_type=jnp.float32),
                                0.0),
                    wout[:, :], preferred_element_type=jnp.float32)
                f = rs_send(s)
                f.start()
                rs_flights.append(f)
            rs_flights_prev = rs_flights
            acc = p_own
            for s in range(3):
                rs_wait_recv(s)
                acc = acc + rsbuf[s]
            for f in ag_flights:
                f.wait_send()
            xown[:, :] = acc
            ag_flights = [ag_send(s) for s in range(3)]
            for f in ag_flights:
                f.start()

        out_ref[pl.ds(my * m_per, m_per), :] = xown[:, :]
        for s in range(3):
            ag_wait_recv(s)
            c = peers[s]
            out_ref[pl.ds(c * m_per, m_per), :] = xin[s]
        for f in ag_flights:
            f.wait_send()
        for f in rs_flights_prev:
            f.wait_send()

        for j in peers:
            pl.semaphore_signal(barrier_sem, inc=1, device_id=(j,),
                                device_id_type=pl.DeviceIdType.MESH)
        pl.semaphore_wait(barrier_sem, N_DEV - 1)

    return pl.pallas_call(
        body,
        out_shape=jax.ShapeDtypeStruct((M, d), jnp.float32),
        in_specs=[pl.BlockSpec(memory_space=pltpu.VMEM)] * 7,
        out_specs=pl.BlockSpec(memory_space=pltpu.VMEM),
        scratch_shapes=[
            pltpu.VMEM((m_per, d), jnp.float32),
            pltpu.VMEM((3, m_per, d), jnp.float32),
            pltpu.VMEM((3, m_per, d), jnp.float32),
            pltpu.VMEM((3, m_per, d), jnp.float32),
            pltpu.SemaphoreType.DMA((3,)),
            pltpu.SemaphoreType.DMA((3,)),
            pltpu.SemaphoreType.DMA((3,)),
            pltpu.SemaphoreType.DMA((3,)),
        ],
        compiler_params=pltpu.CompilerParams(collective_id=0),
    )(x, Win0, Wout0, Win1, Wout1, Win2, Wout2)


# device time: 45276 ns/iter; 2.4016x vs baseline; 1.3758x over previous
import jax
import jax.numpy as jnp
from jax import lax
from jax.experimental import pallas as pl
from jax.experimental.pallas import tpu as pltpu

N_DEV = 4
DST_SLOT = (1, 0, 2)


def kernel(x, Win0, Wout0, Win1, Wout1, Win2, Wout2):
    m_per, d = x.shape
    M = N_DEV * m_per

    def body(x_ref, win0, wout0, win1, wout1, win2, wout2,
             out_ref, xown, xown_bf, xin, pbuf, rsbuf,
             ag_rsems, rs_rsems, ag_ssems, rs_ssems):
        my = lax.axis_index("i")
        peers = ((my + 1) % N_DEV, (my + 3) % N_DEV, (my + 2) % N_DEV)

        barrier_sem = pltpu.get_barrier_semaphore()
        for j in peers:
            pl.semaphore_signal(barrier_sem, inc=1, device_id=(j,),
                                device_id_type=pl.DeviceIdType.MESH)
        pl.semaphore_wait(barrier_sem, N_DEV - 1)

        def ag_send(s):
            return pltpu.make_async_remote_copy(
                src_ref=xown_bf, dst_ref=xin.at[DST_SLOT[s]],
                send_sem=ag_ssems.at[s], recv_sem=ag_rsems.at[DST_SLOT[s]],
                device_id=(peers[s],), device_id_type=pl.DeviceIdType.MESH)

        def ag_wait_recv(s):
            pltpu.make_async_remote_copy(
                src_ref=xown_bf, dst_ref=xin.at[s],
                send_sem=ag_ssems.at[s], recv_sem=ag_rsems.at[s],
                device_id=(peers[s],), device_id_type=pl.DeviceIdType.MESH,
            ).wait_recv()

        def rs_send(s):
            return pltpu.make_async_remote_copy(
                src_ref=pbuf.at[s], dst_ref=rsbuf.at[DST_SLOT[s]],
                send_sem=rs_ssems.at[s], recv_sem=rs_rsems.at[DST_SLOT[s]],
                device_id=(peers[s],), device_id_type=pl.DeviceIdType.MESH)

        def rs_wait_recv(s):
            pltpu.make_async_remote_copy(
                src_ref=pbuf.at[s], dst_ref=rsbuf.at[s],
                send_sem=rs_ssems.at[s], recv_sem=rs_rsems.at[s],
                device_id=(peers[s],), device_id_type=pl.DeviceIdType.MESH,
            ).wait_recv()

        xown[:, :] = x_ref[:, :]
        xown_bf[:, :] = x_ref[:, :].astype(jnp.bfloat16)
        ag_flights = [ag_send(s) for s in range(3)]
        for f in ag_flights:
            f.start()

        rs_flights_prev = None
        for li, (win, wout) in enumerate(((win0, wout0), (win1, wout1),
                                          (win2, wout2))):
            p_own = jnp.dot(
                jnp.maximum(jnp.dot(xown[:, :], win[:, :],
                                    preferred_element_type=jnp.float32), 0.0),
                wout[:, :], preferred_element_type=jnp.float32)
            rs_flights = []
            for s in range(3):
                ag_wait_recv(s)
                if li > 0:
                    rs_flights_prev[s].wait_send()
                pbuf[s] = jnp.dot(
                    jnp.maximum(jnp.dot(xin[s].astype(jnp.float32), win[:, :],
                                        preferred_element_type=jnp.float32),
                                0.0),
                    wout[:, :], preferred_element_type=jnp.float32,
                ).astype(jnp.bfloat16)
                f = rs_send(s)
                f.start()
                rs_flights.append(f)
            rs_flights_prev = rs_flights
            acc = p_own
            for s in range(3):
                rs_wait_recv(s)
                acc = acc + rsbuf[s].astype(jnp.float32)
            for f in ag_flights:
                f.wait_send()
            xown[:, :] = acc
            xown_bf[:, :] = acc.astype(jnp.bfloat16)
            ag_flights = [ag_send(s) for s in range(3)]
            for f in ag_flights:
                f.start()

        out_ref[pl.ds(my * m_per, m_per), :] = xown[:, :]
        for s in range(3):
            ag_wait_recv(s)
            c = peers[s]
            out_ref[pl.ds(c * m_per, m_per), :] = xin[s].astype(jnp.float32)
        for f in ag_flights:
            f.wait_send()
        for f in rs_flights_prev:
            f.wait_send()

        for j in peers:
            pl.semaphore_signal(barrier_sem, inc=1, device_id=(j,),
                                device_id_type=pl.DeviceIdType.MESH)
        pl.semaphore_wait(barrier_sem, N_DEV - 1)

    return pl.pallas_call(
        body,
        out_shape=jax.ShapeDtypeStruct((M, d), jnp.float32),
        in_specs=[pl.BlockSpec(memory_space=pltpu.VMEM)] * 7,
        out_specs=pl.BlockSpec(memory_space=pltpu.VMEM),
        scratch_shapes=[
            pltpu.VMEM((m_per, d), jnp.float32),
            pltpu.VMEM((m_per, d), jnp.bfloat16),
            pltpu.VMEM((3, m_per, d), jnp.bfloat16),
            pltpu.VMEM((3, m_per, d), jnp.bfloat16),
            pltpu.VMEM((3, m_per, d), jnp.bfloat16),
            pltpu.SemaphoreType.DMA((3,)),
            pltpu.SemaphoreType.DMA((3,)),
            pltpu.SemaphoreType.DMA((3,)),
            pltpu.SemaphoreType.DMA((3,)),
        ],
        compiler_params=pltpu.CompilerParams(collective_id=0),
    )(x, Win0, Wout0, Win1, Wout1, Win2, Wout2)
